# baseline (speedup 1.0000x reference)
import functools

import jax
import jax.numpy as jnp
from jax import lax
from jax.experimental import pallas as pl
from jax.experimental.pallas import tpu as pltpu
from jax.experimental.pallas import tpu_sc as plsc

N = 10000
E = 320000
D = 128
NP = 10240
NC = 2
NS = 16
NW = NC * NS
EPW = E // NW
CH = 128
EPW2 = 10240
PH = 2
HSLAB = EPW2 // PH
NCHP = HSLAB // CH
NPAIRP = NCHP // 2
RPT = NP // NS
ZR = 32

_f32 = jnp.float32


@functools.lru_cache(maxsize=None)
def _sc_mesh():
  return plsc.VectorSubcoreMesh(core_axis_name="c", subcore_axis_name="s",
                                num_cores=NC, num_subcores=NS)


def _seg128_body(y_hbm, ei_hbm, a_hbm,
                 src_v, dst_v, dch_a, dch_b, rows_a, rows_b, zb_v,
                 sem_a, sem_b, acc_sh):
  cid = lax.axis_index("c")
  sid = lax.axis_index("s")
  wid = cid * NS + sid

  zeros16 = jnp.zeros((16,), _f32)

  def _z(k, _):
    zb_v[k // 8, pl.ds((k % 8) * 16, 16)] = zeros16
    return 0
  lax.fori_loop(0, ZR * 8, _z, 0)
  for q in range(RPT // ZR):
    pltpu.async_copy(zb_v, acc_sh.at[pl.ds(sid * RPT + q * ZR, ZR), :], sem_a)
  for q in range(RPT // ZR):
    pltpu.make_async_copy(zb_v, acc_sh.at[pl.ds(sid * RPT + q * ZR, ZR), :],
                          sem_a).wait()
  plsc.subcore_barrier()

  def _gather(c, rows, sem):
    return pltpu.async_copy(y_hbm.at[src_v.at[pl.ds(c * CH, CH)]], rows, sem)

  def _wait(c, rows, sem):
    pltpu.make_async_copy(y_hbm.at[src_v.at[pl.ds(c * CH, CH)]],
                          rows, sem).wait()

  def _fill_dch(c, dch):
    for j in range(CH // 16):
      dch[pl.ds(j * 16, 16)] = dst_v[pl.ds(c * CH + j * 16, 16)]

  def _scatter(rows, dch):
    pltpu.sync_copy(rows, acc_sh.at[dch], add=True)

  for ph in range(PH):
    sbase = wid * EPW2 + ph * HSLAB
    dbase = NW * EPW2 + sbase
    pltpu.async_copy(ei_hbm.at[pl.ds(sbase, HSLAB)], src_v, sem_b)
    pltpu.async_copy(ei_hbm.at[pl.ds(dbase, HSLAB)], dst_v, sem_b)
    pltpu.make_async_copy(ei_hbm.at[pl.ds(sbase, HSLAB)], src_v, sem_b).wait()
    pltpu.make_async_copy(ei_hbm.at[pl.ds(dbase, HSLAB)], dst_v, sem_b).wait()

    _gather(0, rows_a, sem_a)
    _gather(1, rows_b, sem_b)

    def _pair_step(g2, _):
      c0 = g2 * 2
      _fill_dch(c0, dch_a)
      _wait(c0, rows_a, sem_a)
      _scatter(rows_a, dch_a)

      @pl.when(g2 + 1 < NPAIRP)
      def _():
        _gather(c0 + 2, rows_a, sem_a)
      _fill_dch(c0 + 1, dch_b)
      _wait(c0 + 1, rows_b, sem_b)
      _scatter(rows_b, dch_b)

      @pl.when(g2 + 1 < NPAIRP)
      def _():
        _gather(c0 + 3, rows_b, sem_b)
      return 0

    lax.fori_loop(0, NPAIRP, _pair_step, 0)
  plsc.subcore_barrier()

  pltpu.sync_copy(acc_sh.at[pl.ds(sid * RPT, RPT), :],
                  a_hbm.at[cid, pl.ds(sid * RPT, RPT), :])


@functools.lru_cache(maxsize=None)
def _seg128():
  return pl.kernel(
      _seg128_body,
      out_type=(jax.ShapeDtypeStruct((NC, NP, D), _f32),),
      mesh=_sc_mesh(),
      scratch_types=[
          pltpu.VMEM((HSLAB,), jnp.int32),
          pltpu.VMEM((HSLAB,), jnp.int32),
          pltpu.VMEM((CH,), jnp.int32),
          pltpu.VMEM((CH,), jnp.int32),
          pltpu.VMEM((CH, D), _f32),
          pltpu.VMEM((CH, D), _f32),
          pltpu.VMEM((ZR, D), _f32),
          pltpu.SemaphoreType.DMA,
          pltpu.SemaphoreType.DMA,
          pltpu.VMEM_SHARED((NP, D), _f32),
      ],
      compiler_params=pltpu.CompilerParams(needs_layout_passes=False),
  )


def _cnt_body(ei_hbm, cnt_hbm, dst_v, cnt_v, tmp_v, cnt_sh):
  cid = lax.axis_index("c")
  sid = lax.axis_index("s")
  wid = cid * NS + sid

  ones16 = jnp.ones((16,), _f32)

  def _zc(k, _):
    cnt_v[pl.ds(k * 16, 16)] = jnp.zeros((16,), _f32)
    return 0
  lax.fori_loop(0, NP // 16, _zc, 0)

  pltpu.sync_copy(ei_hbm.at[pl.ds(E + wid * EPW, EPW)], dst_v)

  def _step(j, _):
    plsc.addupdate_scatter(cnt_v, [dst_v[pl.ds(j * 16, 16)]], ones16)
    return 0
  lax.fori_loop(0, EPW // 16, _step, 0)

  pltpu.sync_copy(cnt_v, cnt_sh.at[sid])
  plsc.subcore_barrier()

  def _zr(g, _):
    cnt_v[pl.ds(g * 16, 16)] = jnp.zeros((16,), _f32)
    return 0
  lax.fori_loop(0, RPT // 16, _zr, 0)
  for t in range(NS):
    pltpu.sync_copy(cnt_sh.at[t, pl.ds(sid * RPT, RPT)], tmp_v)
    def _acc(g, _):
      cnt_v[pl.ds(g * 16, 16)] = (cnt_v[pl.ds(g * 16, 16)]
                                  + tmp_v[pl.ds(g * 16, 16)])
      return 0
    lax.fori_loop(0, RPT // 16, _acc, 0)
  pltpu.sync_copy(cnt_v.at[pl.ds(0, RPT)],
                  cnt_hbm.at[pl.ds(cid * NP + sid * RPT, RPT)])


@functools.lru_cache(maxsize=None)
def _cnt():
  return pl.kernel(
      _cnt_body,
      out_type=(jax.ShapeDtypeStruct((NC * NP,), _f32),),
      mesh=_sc_mesh(),
      scratch_types=[
          pltpu.VMEM((EPW,), jnp.int32),
          pltpu.VMEM((NP,), _f32),
          pltpu.VMEM((RPT,), _f32),
          pltpu.VMEM_SHARED((NS, NP), _f32),
      ],
      compiler_params=pltpu.CompilerParams(needs_layout_passes=False),
  )


def _pair_body(u_hbm, ei_hbm, pa_hbm, pc_hbm,
               ua_v, uc_v, acca_v, accc_v, src_v, dst_v):
  cid = lax.axis_index("c")
  sid = lax.axis_index("s")
  wid = cid * NS + sid

  zeros16 = jnp.zeros((16,), _f32)

  def _z(k, _):
    acca_v[pl.ds(k * 16, 16)] = zeros16
    accc_v[pl.ds(k * 16, 16)] = zeros16
    return 0
  lax.fori_loop(0, NP // 16, _z, 0)

  pltpu.sync_copy(u_hbm.at[pl.ds(0, NP)], ua_v)
  pltpu.sync_copy(u_hbm.at[pl.ds(NP, NP)], uc_v)
  pltpu.sync_copy(ei_hbm.at[pl.ds(wid * EPW, EPW)], src_v)
  pltpu.sync_copy(ei_hbm.at[pl.ds(E + wid * EPW, EPW)], dst_v)

  def _step(j, _):
    s16 = src_v[pl.ds(j * 16, 16)]
    d16 = dst_v[pl.ds(j * 16, 16)]
    plsc.addupdate_scatter(acca_v, [d16], plsc.load_gather(ua_v, [s16]))
    plsc.addupdate_scatter(accc_v, [d16], plsc.load_gather(uc_v, [s16]))
    return 0
  lax.fori_loop(0, EPW // 16, _step, 0)

  pltpu.sync_copy(acca_v, pa_hbm.at[pl.ds(wid * NP, NP)])
  pltpu.sync_copy(accc_v, pc_hbm.at[pl.ds(wid * NP, NP)])


@functools.lru_cache(maxsize=None)
def _pair():
  return pl.kernel(
      _pair_body,
      out_type=(jax.ShapeDtypeStruct((NW * NP,), _f32),
                jax.ShapeDtypeStruct((NW * NP,), _f32)),
      mesh=_sc_mesh(),
      scratch_types=[
          pltpu.VMEM((NP,), _f32),
          pltpu.VMEM((NP,), _f32),
          pltpu.VMEM((NP,), _f32),
          pltpu.VMEM((NP,), _f32),
          pltpu.VMEM((EPW,), jnp.int32),
          pltpu.VMEM((EPW,), jnp.int32),
      ],
      compiler_params=pltpu.CompilerParams(needs_layout_passes=False),
  )


def _dotT(a, b):
  return lax.dot_general(a, b, (((1,), (1,)), ((), ())),
                         preferred_element_type=_f32)


def _tc0_body(x_ref, wl_ref, wr_ref, bl_ref, y_ref, r_ref):
  x = x_ref[...]
  y_ref[...] = _dotT(x, wl_ref[...])
  r_ref[...] = _dotT(x, wr_ref[...]) + bl_ref[...]


def _tc1_body(a_ref, cm_ref, r0_ref, wl_ref, wr_ref, bl_ref,
              y_ref, r_ref):
  h = jnp.tanh((a_ref[0] + a_ref[1]) / cm_ref[...] + r0_ref[...])
  y_ref[...] = _dotT(h, wl_ref[...])
  r_ref[...] = _dotT(h, wr_ref[...]) + bl_ref[...]


def _tc2_body(a_ref, cm_ref, r1_ref, wl2_ref, wa_ref, wl3_ref, wc_ref,
              wr2_ref, wr3_ref, u_ref):
  h1 = jnp.tanh((a_ref[0] + a_ref[1]) / cm_ref[...] + r1_ref[...])
  ga = lax.dot_general(wl2_ref[...], wa_ref[...], (((0,), (1,)), ((), ())),
                       preferred_element_type=_f32)
  gc = lax.dot_general(wl3_ref[...], wc_ref[...], (((0,), (1,)), ((), ())),
                       preferred_element_type=_f32)
  ra = lax.dot_general(wr2_ref[...], wa_ref[...], (((0,), (1,)), ((), ())),
                       preferred_element_type=_f32)
  rc = lax.dot_general(wr3_ref[...], wc_ref[...], (((0,), (1,)), ((), ())),
                       preferred_element_type=_f32)
  g4 = jnp.concatenate([ga, gc, ra, rc], axis=1)
  u_ref[...] = lax.dot_general(g4, h1, (((0,), (1,)), ((), ())),
                               preferred_element_type=_f32)


def _tc3_body(pa_ref, pc_ref, cm_ref, u_ref, bl3_ref, wc_ref, bc_ref,
              i_ref, ea_ref, ec_ref):
  iota = lax.broadcasted_iota(jnp.int32, (1, NP), 1)
  valid = iota < N
  cm = cm_ref[...]
  sa = jnp.sum(pa_ref[...], axis=0, keepdims=True)
  la = u_ref[2:3, :]
  ta = 0.5 * (sa / cm + la)
  ta = jnp.where(valid, ta, -1e30)
  m = jnp.max(ta)
  lse = jnp.log(jnp.sum(jnp.exp(ta - m))) + m
  ea_ref[...] = ta - lse

  sc = jnp.sum(pc_ref[...], axis=0, keepdims=True)
  lc = u_ref[3:4, :]
  cc = jnp.sum(bl3_ref[...] * wc_ref[0, :])
  tcv = sc / cm + lc + cc
  idx = i_ref[0]
  tci = jnp.sum(jnp.where(iota == idx, tcv, 0.0))
  mean_tc = jnp.sum(jnp.where(valid, tcv, 0.0)) / N
  val = 0.5 * tci + 0.5 * mean_tc + bc_ref[0]
  ec_ref[...] = jnp.tanh(val).reshape(1, 1)


def _tc_call(body, n_in, out_shape, smem_arg=None):
  in_specs = [pl.BlockSpec(memory_space=pltpu.VMEM) for _ in range(n_in)]
  if smem_arg is not None:
    in_specs[smem_arg] = pl.BlockSpec(memory_space=pltpu.SMEM)
  return pl.pallas_call(
      body,
      out_shape=out_shape,
      in_specs=in_specs,
      out_specs=tuple(pl.BlockSpec(memory_space=pltpu.VMEM)
                      for _ in range(len(out_shape))),
  )


def kernel(x, edge_index, i, W_l0, b_l0, W_r0, W_l1, b_l1, W_r1,
           W_l2, b_l2, W_r2, W_l3, b_l3, W_r3, Wa, ba, Wc, bc):
  del b_l2, ba

  xp = jnp.pad(x, ((0, NP - N), (0, 0)))
  ei = edge_index.reshape(-1)
  epad = jnp.concatenate(
      [edge_index.reshape(2, NW, EPW),
       jnp.full((2, NW, EPW2 - EPW), N, jnp.int32)], axis=2).reshape(-1)

  nd = jax.ShapeDtypeStruct
  y0, r0 = _tc_call(_tc0_body, 4,
                    (nd((NP, D), _f32), nd((NP, D), _f32)))(
                        xp, W_l0, W_r0, b_l0)

  cntf = _cnt()(ei)[0]
  a0 = _seg128()(y0, epad)[0]

  cm = jnp.maximum(cntf[:NP] + cntf[NP:], 1.0)
  cm_col = cm.reshape(NP, 1)
  cm_row = cm.reshape(1, NP)

  y1, r1 = _tc_call(_tc1_body, 6,
                    (nd((NP, D), _f32), nd((NP, D), _f32)))(
                        a0, cm_col, r0, W_l1, W_r1, b_l1)

  a1 = _seg128()(y1, epad)[0]

  u = _tc_call(_tc2_body, 9, (nd((4, NP), _f32),))(
      a1, cm_col, r1, W_l2, Wa, W_l3, Wc, W_r2, W_r3)[0]

  pa, pc = _pair()(u.reshape(-1), ei)

  i_arr = jnp.asarray(i, jnp.int32).reshape(1)
  ea, ec = _tc_call(_tc3_body, 8,
                    (nd((1, NP), _f32), nd((1, 1), _f32)),
                    smem_arg=7)(
                        pa.reshape(NW, NP), pc.reshape(NW, NP),
                        cm_row, u, b_l3, Wc, bc, i_arr)
  return ea[0, :N].reshape(N, 1), ec


# speedup vs baseline: 2.7218x; 2.7218x over previous
import functools

import jax
import jax.numpy as jnp
from jax import lax
from jax.experimental import pallas as pl
from jax.experimental.pallas import tpu as pltpu
from jax.experimental.pallas import tpu_sc as plsc

N = 10000
E = 320000
D = 128
NP = 10240
NC = 2
NS = 16
NW = NC * NS
EPW = E // NW
CH = 80
NCHUNK = EPW // CH
NPAIR = (NCHUNK - 1) // 2
RPT = NP // NS
ZR = 32

_f32 = jnp.float32


@functools.lru_cache(maxsize=None)
def _sc_mesh():
  return plsc.VectorSubcoreMesh(core_axis_name="c", subcore_axis_name="s",
                                num_cores=NC, num_subcores=NS)


def _seg128_body(y_hbm, ei_hbm, a_hbm,
                 src_v, dst_v, dch_a, dch_b, rows_a, rows_b, zb_v,
                 sem_a, sem_b, acc_sh):
  cid = lax.axis_index("c")
  sid = lax.axis_index("s")
  wid = cid * NS + sid

  zeros16 = jnp.zeros((16,), _f32)

  def _z(k, _):
    zb_v[k // 8, pl.ds((k % 8) * 16, 16)] = zeros16
    return 0
  lax.fori_loop(0, ZR * 8, _z, 0)
  for q in range(RPT // ZR):
    pltpu.async_copy(zb_v, acc_sh.at[pl.ds(sid * RPT + q * ZR, ZR), :], sem_a)
  pltpu.async_copy(ei_hbm.at[pl.ds(wid * EPW, EPW)], src_v, sem_b)
  pltpu.async_copy(ei_hbm.at[pl.ds(E + wid * EPW, EPW)], dst_v, sem_b)
  for q in range(RPT // ZR):
    pltpu.make_async_copy(zb_v, acc_sh.at[pl.ds(sid * RPT + q * ZR, ZR), :],
                          sem_a).wait()
  pltpu.make_async_copy(ei_hbm.at[pl.ds(wid * EPW, EPW)], src_v, sem_b).wait()
  pltpu.make_async_copy(ei_hbm.at[pl.ds(E + wid * EPW, EPW)], dst_v,
                        sem_b).wait()
  plsc.subcore_barrier()

  def _gather(c, rows, sem):
    return pltpu.async_copy(y_hbm.at[src_v.at[pl.ds(c * CH, CH)]], rows, sem)

  def _wait(c, rows, sem):
    pltpu.make_async_copy(y_hbm.at[src_v.at[pl.ds(c * CH, CH)]],
                          rows, sem).wait()

  def _fill_dch(c, dch):
    for j in range(CH // 16):
      dch[pl.ds(j * 16, 16)] = dst_v[pl.ds(c * CH + j * 16, 16)]

  def _scatter(rows, dch):
    pltpu.sync_copy(rows, acc_sh.at[dch], add=True)

  _gather(0, rows_a, sem_a)

  def _pair_step(g2, _):
    c0 = g2 * 2
    _gather(c0 + 1, rows_b, sem_b)
    _fill_dch(c0, dch_a)
    _wait(c0, rows_a, sem_a)
    _scatter(rows_a, dch_a)
    _gather(c0 + 2, rows_a, sem_a)
    _fill_dch(c0 + 1, dch_b)
    _wait(c0 + 1, rows_b, sem_b)
    _scatter(rows_b, dch_b)
    return 0

  lax.fori_loop(0, NPAIR, _pair_step, 0)
  _fill_dch(NCHUNK - 1, dch_a)
  _wait(NCHUNK - 1, rows_a, sem_a)
  _scatter(rows_a, dch_a)
  plsc.subcore_barrier()

  pltpu.sync_copy(acc_sh.at[pl.ds(sid * RPT, RPT), :],
                  a_hbm.at[cid, pl.ds(sid * RPT, RPT), :])


@functools.lru_cache(maxsize=None)
def _seg128():
  return pl.kernel(
      _seg128_body,
      out_type=(jax.ShapeDtypeStruct((NC, NP, D), _f32),),
      mesh=_sc_mesh(),
      scratch_types=[
          pltpu.VMEM((EPW,), jnp.int32),
          pltpu.VMEM((EPW,), jnp.int32),
          pltpu.VMEM((CH,), jnp.int32),
          pltpu.VMEM((CH,), jnp.int32),
          pltpu.VMEM((CH, D), _f32),
          pltpu.VMEM((CH, D), _f32),
          pltpu.VMEM((ZR, D), _f32),
          pltpu.SemaphoreType.DMA,
          pltpu.SemaphoreType.DMA,
          pltpu.VMEM_SHARED((NP, D), _f32),
      ],
      compiler_params=pltpu.CompilerParams(needs_layout_passes=False),
  )


def _cnt_body(ei_hbm, cnt_hbm, dst_v, cnt_v, tmp_v, cnt_sh):
  cid = lax.axis_index("c")
  sid = lax.axis_index("s")
  wid = cid * NS + sid

  ones16 = jnp.ones((16,), _f32)

  def _zc(k, _):
    cnt_v[pl.ds(k * 16, 16)] = jnp.zeros((16,), _f32)
    return 0
  lax.fori_loop(0, NP // 16, _zc, 0)

  pltpu.sync_copy(ei_hbm.at[pl.ds(E + wid * EPW, EPW)], dst_v)

  def _step(j, _):
    plsc.addupdate_scatter(cnt_v, [dst_v[pl.ds(j * 16, 16)]], ones16)
    return 0
  lax.fori_loop(0, EPW // 16, _step, 0)

  pltpu.sync_copy(cnt_v, cnt_sh.at[sid])
  plsc.subcore_barrier()

  def _zr(g, _):
    cnt_v[pl.ds(g * 16, 16)] = jnp.zeros((16,), _f32)
    return 0
  lax.fori_loop(0, RPT // 16, _zr, 0)
  for t in range(NS):
    pltpu.sync_copy(cnt_sh.at[t, pl.ds(sid * RPT, RPT)], tmp_v)
    def _acc(g, _):
      cnt_v[pl.ds(g * 16, 16)] = (cnt_v[pl.ds(g * 16, 16)]
                                  + tmp_v[pl.ds(g * 16, 16)])
      return 0
    lax.fori_loop(0, RPT // 16, _acc, 0)
  pltpu.sync_copy(cnt_v.at[pl.ds(0, RPT)],
                  cnt_hbm.at[pl.ds(cid * NP + sid * RPT, RPT)])


@functools.lru_cache(maxsize=None)
def _cnt():
  return pl.kernel(
      _cnt_body,
      out_type=(jax.ShapeDtypeStruct((NC * NP,), _f32),),
      mesh=_sc_mesh(),
      scratch_types=[
          pltpu.VMEM((EPW,), jnp.int32),
          pltpu.VMEM((NP,), _f32),
          pltpu.VMEM((RPT,), _f32),
          pltpu.VMEM_SHARED((NS, NP), _f32),
      ],
      compiler_params=pltpu.CompilerParams(needs_layout_passes=False),
  )


def _pair_body(u_hbm, ei_hbm, pa_hbm, pc_hbm,
               ua_v, uc_v, acca_v, accc_v, src_v, dst_v):
  cid = lax.axis_index("c")
  sid = lax.axis_index("s")
  wid = cid * NS + sid

  zeros16 = jnp.zeros((16,), _f32)

  def _z(k, _):
    acca_v[pl.ds(k * 16, 16)] = zeros16
    accc_v[pl.ds(k * 16, 16)] = zeros16
    return 0
  lax.fori_loop(0, NP // 16, _z, 0)

  pltpu.sync_copy(u_hbm.at[pl.ds(0, NP)], ua_v)
  pltpu.sync_copy(u_hbm.at[pl.ds(NP, NP)], uc_v)
  pltpu.sync_copy(ei_hbm.at[pl.ds(wid * EPW, EPW)], src_v)
  pltpu.sync_copy(ei_hbm.at[pl.ds(E + wid * EPW, EPW)], dst_v)

  def _step(j, _):
    s16 = src_v[pl.ds(j * 16, 16)]
    d16 = dst_v[pl.ds(j * 16, 16)]
    plsc.addupdate_scatter(acca_v, [d16], plsc.load_gather(ua_v, [s16]))
    plsc.addupdate_scatter(accc_v, [d16], plsc.load_gather(uc_v, [s16]))
    return 0
  lax.fori_loop(0, EPW // 16, _step, 0)

  pltpu.sync_copy(acca_v, pa_hbm.at[pl.ds(wid * NP, NP)])
  pltpu.sync_copy(accc_v, pc_hbm.at[pl.ds(wid * NP, NP)])


@functools.lru_cache(maxsize=None)
def _pair():
  return pl.kernel(
      _pair_body,
      out_type=(jax.ShapeDtypeStruct((NW * NP,), _f32),
                jax.ShapeDtypeStruct((NW * NP,), _f32)),
      mesh=_sc_mesh(),
      scratch_types=[
          pltpu.VMEM((NP,), _f32),
          pltpu.VMEM((NP,), _f32),
          pltpu.VMEM((NP,), _f32),
          pltpu.VMEM((NP,), _f32),
          pltpu.VMEM((EPW,), jnp.int32),
          pltpu.VMEM((EPW,), jnp.int32),
      ],
      compiler_params=pltpu.CompilerParams(needs_layout_passes=False),
  )


def _dotT(a, b):
  return lax.dot_general(a, b, (((1,), (1,)), ((), ())),
                         preferred_element_type=_f32)


def _tc0_body(x_ref, wl_ref, wr_ref, bl_ref, y_ref, r_ref):
  x = x_ref[...]
  y_ref[...] = _dotT(x, wl_ref[...])
  r_ref[...] = _dotT(x, wr_ref[...]) + bl_ref[...]


def _tc1_body(a_ref, cm_ref, r0_ref, wl_ref, wr_ref, bl_ref,
              y_ref, r_ref):
  h = jnp.tanh((a_ref[0] + a_ref[1]) / cm_ref[...] + r0_ref[...])
  y_ref[...] = _dotT(h, wl_ref[...])
  r_ref[...] = _dotT(h, wr_ref[...]) + bl_ref[...]


def _tc2_body(a_ref, cm_ref, r1_ref, wl2_ref, wa_ref, wl3_ref, wc_ref,
              wr2_ref, wr3_ref, u_ref):
  h1 = jnp.tanh((a_ref[0] + a_ref[1]) / cm_ref[...] + r1_ref[...])
  ga = lax.dot_general(wl2_ref[...], wa_ref[...], (((0,), (1,)), ((), ())),
                       preferred_element_type=_f32)
  gc = lax.dot_general(wl3_ref[...], wc_ref[...], (((0,), (1,)), ((), ())),
                       preferred_element_type=_f32)
  ra = lax.dot_general(wr2_ref[...], wa_ref[...], (((0,), (1,)), ((), ())),
                       preferred_element_type=_f32)
  rc = lax.dot_general(wr3_ref[...], wc_ref[...], (((0,), (1,)), ((), ())),
                       preferred_element_type=_f32)
  g4 = jnp.concatenate([ga, gc, ra, rc], axis=1)
  u_ref[...] = lax.dot_general(g4, h1, (((0,), (1,)), ((), ())),
                               preferred_element_type=_f32)


def _tc3_body(pa_ref, pc_ref, cm_ref, u_ref, bl3_ref, wc_ref, bc_ref,
              i_ref, ea_ref, ec_ref):
  iota = lax.broadcasted_iota(jnp.int32, (1, NP), 1)
  valid = iota < N
  cm = cm_ref[...]
  sa = jnp.sum(pa_ref[...], axis=0, keepdims=True)
  la = u_ref[2:3, :]
  ta = 0.5 * (sa / cm + la)
  ta = jnp.where(valid, ta, -1e30)
  m = jnp.max(ta)
  lse = jnp.log(jnp.sum(jnp.exp(ta - m))) + m
  ea_ref[...] = ta - lse

  sc = jnp.sum(pc_ref[...], axis=0, keepdims=True)
  lc = u_ref[3:4, :]
  cc = jnp.sum(bl3_ref[...] * wc_ref[0, :])
  tcv = sc / cm + lc + cc
  idx = i_ref[0]
  tci = jnp.sum(jnp.where(iota == idx, tcv, 0.0))
  mean_tc = jnp.sum(jnp.where(valid, tcv, 0.0)) / N
  val = 0.5 * tci + 0.5 * mean_tc + bc_ref[0]
  ec_ref[...] = jnp.tanh(val).reshape(1, 1)


def _tc_call(body, n_in, out_shape, smem_arg=None):
  in_specs = [pl.BlockSpec(memory_space=pltpu.VMEM) for _ in range(n_in)]
  if smem_arg is not None:
    in_specs[smem_arg] = pl.BlockSpec(memory_space=pltpu.SMEM)
  return pl.pallas_call(
      body,
      out_shape=out_shape,
      in_specs=in_specs,
      out_specs=tuple(pl.BlockSpec(memory_space=pltpu.VMEM)
                      for _ in range(len(out_shape))),
  )


def kernel(x, edge_index, i, W_l0, b_l0, W_r0, W_l1, b_l1, W_r1,
           W_l2, b_l2, W_r2, W_l3, b_l3, W_r3, Wa, ba, Wc, bc):
  del b_l2, ba

  xp = jnp.pad(x, ((0, NP - N), (0, 0)))
  ei = edge_index.reshape(-1)

  nd = jax.ShapeDtypeStruct
  y0, r0 = _tc_call(_tc0_body, 4,
                    (nd((NP, D), _f32), nd((NP, D), _f32)))(
                        xp, W_l0, W_r0, b_l0)

  cntf = _cnt()(ei)[0]
  a0 = _seg128()(y0, ei)[0]

  cm = jnp.maximum(cntf[:NP] + cntf[NP:], 1.0)
  cm_col = cm.reshape(NP, 1)
  cm_row = cm.reshape(1, NP)

  y1, r1 = _tc_call(_tc1_body, 6,
                    (nd((NP, D), _f32), nd((NP, D), _f32)))(
                        a0, cm_col, r0, W_l1, W_r1, b_l1)

  a1 = _seg128()(y1, ei)[0]

  u = _tc_call(_tc2_body, 9, (nd((4, NP), _f32),))(
      a1, cm_col, r1, W_l2, Wa, W_l3, Wc, W_r2, W_r3)[0]

  pa, pc = _pair()(u.reshape(-1), ei)

  i_arr = jnp.asarray(i, jnp.int32).reshape(1)
  ea, ec = _tc_call(_tc3_body, 8,
                    (nd((1, NP), _f32), nd((1, 1), _f32)),
                    smem_arg=7)(
                        pa.reshape(NW, NP), pc.reshape(NW, NP),
                        cm_row, u, b_l3, Wc, bc, i_arr)
  return ea[0, :N].reshape(N, 1), ec
